# speedup vs baseline: 3.0356x; 3.0356x over previous
import functools

import jax
import jax.numpy as jnp
from jax.experimental import pallas as pl
from jax.experimental.pallas import tpu as pltpu
from jax.experimental.pallas import tpu_sc as plsc

_W = 128
_K = 2


def _gather_pipeline(table_hbm, ids_hbm, out_hbm):
    n_rows = out_hbm.shape[0]
    d = out_hbm.shape[1]

    def body(i_vmem, o_vmem):
        for j in range(_K):
            pltpu.sync_copy(
                table_hbm.at[i_vmem.at[j]],
                o_vmem.at[pl.ds(j * _W, _W), :],
            )

    pltpu.emit_pipeline(
        body,
        grid=(n_rows // (_K * _W),),
        in_specs=[pl.BlockSpec((_K, _W), index_map=lambda i: (i, 0))],
        out_specs=[pl.BlockSpec((_K * _W, d), index_map=lambda i: (i, 0))],
        core_axis_name=("c", "s"),
        dimension_semantics=(pltpu.PARALLEL,),
    )(ids_hbm, out_hbm)


def kernel(input_ids, r_table, g_table, b_table):
    b, t, c = input_ids.shape
    v, d = r_table.shape
    n = b * t * c

    table = jnp.concatenate([r_table, g_table, b_table], axis=0)
    offsets = jnp.arange(c, dtype=input_ids.dtype) * v
    flat_ids = (input_ids + offsets).reshape(n // _W, _W)

    mesh = plsc.VectorSubcoreMesh(core_axis_name="c", subcore_axis_name="s")
    gather = pl.kernel(
        _gather_pipeline,
        out_type=jax.ShapeDtypeStruct((n, d), jnp.float32),
        mesh=mesh,
    )
    out = gather(table, flat_ids)
    return out.reshape(b, t, c * d)
